# Initial kernel scaffold; baseline (speedup 1.0000x reference)
import jax
import jax.numpy as jnp
from jax.experimental import pallas as pl


def kernel(x, Wq, Wk, Wv, Wo, hilbert_map):
    raise NotImplementedError("write your pallas kernel here")


# speedup vs baseline: 3.9764x; 3.9764x over previous
import functools
import math

import jax
import jax.numpy as jnp
from jax import lax
from jax.experimental import pallas as pl
from jax.experimental.pallas import tpu as pltpu
from jax.experimental.pallas import tpu_sc as plsc

_NUM_HEADS = 12
_SEGMENT_SIZE = 512
_DILATION = 2

_SC_CORES = 2
_SC_SUBCORES = 16
_SC_WORKERS = _SC_CORES * _SC_SUBCORES


def _sc_gather_rows(table, idx):
    _, d = table.shape
    b = idx.shape[0]
    assert b % (8 * _SC_WORKERS) == 0
    b_per_w = b // _SC_WORKERS
    mesh = plsc.VectorSubcoreMesh(core_axis_name="c", subcore_axis_name="s")

    @functools.partial(
        pl.kernel,
        mesh=mesh,
        out_type=jax.ShapeDtypeStruct((b, d), table.dtype),
        scratch_types=[
            pltpu.VMEM((b_per_w,), jnp.int32),
            pltpu.VMEM((b_per_w, d), table.dtype),
            pltpu.SemaphoreType.DMA,
        ],
    )
    def k(table_hbm, idx_hbm, out_hbm, idx_v, rows_v, sem):
        wid = lax.axis_index("s") * _SC_CORES + lax.axis_index("c")
        base = wid * b_per_w
        pltpu.sync_copy(idx_hbm.at[pl.ds(base, b_per_w)], idx_v)
        pltpu.async_copy(table_hbm.at[idx_v], rows_v, sem).wait()
        pltpu.sync_copy(rows_v, out_hbm.at[pl.ds(base, b_per_w)])

    return k(table, idx)


def _sc_scatter_rows(rows, idx):
    b, d = rows.shape
    assert b % (8 * _SC_WORKERS) == 0
    b_per_w = b // _SC_WORKERS
    mesh = plsc.VectorSubcoreMesh(core_axis_name="c", subcore_axis_name="s")

    @functools.partial(
        pl.kernel,
        mesh=mesh,
        out_type=jax.ShapeDtypeStruct((b, d), rows.dtype),
        scratch_types=[
            pltpu.VMEM((b_per_w,), jnp.int32),
            pltpu.VMEM((b_per_w, d), rows.dtype),
            pltpu.SemaphoreType.DMA,
        ],
    )
    def k(rows_hbm, idx_hbm, out_hbm, idx_v, rows_v, sem):
        wid = lax.axis_index("s") * _SC_CORES + lax.axis_index("c")
        base = wid * b_per_w
        pltpu.sync_copy(idx_hbm.at[pl.ds(base, b_per_w)], idx_v)
        pltpu.sync_copy(rows_hbm.at[pl.ds(base, b_per_w)], rows_v)
        pltpu.async_copy(rows_v, out_hbm.at[idx_v], sem).wait()

    return k(rows, idx)


def _attention_body(x_ref, xg_ref, wq_ref, wk_ref, wv_ref, wo_ref, o_ref,
                    kg_s, vg_s, *, heads, dh, scale):
    qb = pl.program_id(0)

    @pl.when(qb == 0)
    def _():
        kg_s[...] = jnp.dot(xg_ref[...], wk_ref[...],
                            preferred_element_type=jnp.float32)
        vg_s[...] = jnp.dot(xg_ref[...], wv_ref[...],
                            preferred_element_type=jnp.float32)

    q_blk = jnp.dot(x_ref[...], wq_ref[...],
                    preferred_element_type=jnp.float32)
    ctx_parts = []
    for h in range(heads):
        lo = h * dh
        qh = q_blk[:, lo:lo + dh]
        kh = kg_s[:, lo:lo + dh]
        vh = vg_s[:, lo:lo + dh]
        s = lax.dot_general(qh, kh, (((1,), (1,)), ((), ())),
                            preferred_element_type=jnp.float32) * scale
        m = jnp.max(s, axis=-1, keepdims=True)
        e = jnp.exp(s - m)
        p = e / jnp.sum(e, axis=-1, keepdims=True)
        ctx_parts.append(jnp.dot(p, vh, preferred_element_type=jnp.float32))
    ctx = jnp.concatenate(ctx_parts, axis=1)
    o_ref[...] = jnp.dot(ctx, wo_ref[...], preferred_element_type=jnp.float32)


def _tc_attention(x2, xg, wq, wk, wv, wo, heads, dh):
    s_len, d = x2.shape
    nk = xg.shape[0]
    qb_rows = 512
    n_qb = s_len // qb_rows
    scale = 1.0 / math.sqrt(dh)
    body = functools.partial(_attention_body, heads=heads, dh=dh, scale=scale)
    return pl.pallas_call(
        body,
        grid=(n_qb,),
        in_specs=[
            pl.BlockSpec((qb_rows, d), lambda i: (i, 0)),
            pl.BlockSpec((nk, d), lambda i: (0, 0)),
            pl.BlockSpec((d, d), lambda i: (0, 0)),
            pl.BlockSpec((d, d), lambda i: (0, 0)),
            pl.BlockSpec((d, d), lambda i: (0, 0)),
            pl.BlockSpec((d, d), lambda i: (0, 0)),
        ],
        out_specs=pl.BlockSpec((qb_rows, d), lambda i: (i, 0)),
        out_shape=jax.ShapeDtypeStruct((s_len, d), jnp.float32),
        scratch_shapes=[
            pltpu.VMEM((nk, d), jnp.float32),
            pltpu.VMEM((nk, d), jnp.float32),
        ],
        compiler_params=pltpu.CompilerParams(
            dimension_semantics=("arbitrary",)),
    )(x2, xg, wq, wk, wv, wo)


def kernel(x, Wq, Wk, Wv, Wo, hilbert_map):
    b, s_len, d = x.shape
    heads = _NUM_HEADS
    dh = d // heads
    x2 = x.reshape(s_len, d)
    kv_pos = lax.slice(hilbert_map, (0,), (s_len,), (_DILATION,))
    xg = _sc_gather_rows(x2, kv_pos)
    y = _tc_attention(x2, xg, Wq, Wk, Wv, Wo, heads, dh)
    out2 = _sc_scatter_rows(y, hilbert_map)
    return out2.reshape(b, s_len, d)


# speedup vs baseline: 4.8537x; 1.2206x over previous
import functools
import math

import jax
import jax.numpy as jnp
from jax import lax
from jax.experimental import pallas as pl
from jax.experimental.pallas import tpu as pltpu
from jax.experimental.pallas import tpu_sc as plsc

_NUM_HEADS = 12
_SEGMENT_SIZE = 512
_DILATION = 2

_SC_CORES = 2
_SC_SUBCORES = 16
_SC_WORKERS = _SC_CORES * _SC_SUBCORES


def _sc_gather_rows(table, idx):
    _, d = table.shape
    b = idx.shape[0]
    assert b % (8 * _SC_WORKERS) == 0
    b_per_w = b // _SC_WORKERS
    mesh = plsc.VectorSubcoreMesh(core_axis_name="c", subcore_axis_name="s")

    @functools.partial(
        pl.kernel,
        mesh=mesh,
        out_type=jax.ShapeDtypeStruct((b, d), table.dtype),
        scratch_types=[
            pltpu.VMEM((b_per_w,), jnp.int32),
            pltpu.VMEM((b_per_w, d), table.dtype),
            pltpu.SemaphoreType.DMA,
        ],
    )
    def k(table_hbm, idx_hbm, out_hbm, idx_v, rows_v, sem):
        wid = lax.axis_index("s") * _SC_CORES + lax.axis_index("c")
        base = wid * b_per_w
        pltpu.sync_copy(idx_hbm.at[pl.ds(base, b_per_w)], idx_v)
        pltpu.async_copy(table_hbm.at[idx_v], rows_v, sem).wait()
        pltpu.sync_copy(rows_v, out_hbm.at[pl.ds(base, b_per_w)])

    return k(table, idx)


def _sc_scatter_rows(rows, idx):
    b, d = rows.shape
    assert b % (8 * _SC_WORKERS) == 0
    b_per_w = b // _SC_WORKERS
    mesh = plsc.VectorSubcoreMesh(core_axis_name="c", subcore_axis_name="s")

    @functools.partial(
        pl.kernel,
        mesh=mesh,
        out_type=jax.ShapeDtypeStruct((b, d), rows.dtype),
        scratch_types=[
            pltpu.VMEM((b_per_w,), jnp.int32),
            pltpu.VMEM((b_per_w, d), rows.dtype),
            pltpu.SemaphoreType.DMA,
        ],
    )
    def k(rows_hbm, idx_hbm, out_hbm, idx_v, rows_v, sem):
        wid = lax.axis_index("s") * _SC_CORES + lax.axis_index("c")
        base = wid * b_per_w
        pltpu.sync_copy(idx_hbm.at[pl.ds(base, b_per_w)], idx_v)
        pltpu.sync_copy(rows_hbm.at[pl.ds(base, b_per_w)], rows_v)
        pltpu.async_copy(rows_v, out_hbm.at[idx_v], sem).wait()

    return k(rows, idx)


def _proj_body(x_ref, xg_ref, wq_ref, wk_ref, wv_ref, q_ref, kg_ref, vg_ref):
    q_ref[...] = jnp.dot(x_ref[...], wq_ref[...],
                         preferred_element_type=jnp.float32
                         ).astype(jnp.bfloat16)
    xg_bf = xg_ref[...].astype(jnp.bfloat16)
    kg_ref[...] = jnp.dot(xg_bf, wk_ref[...],
                          preferred_element_type=jnp.float32
                          ).astype(jnp.bfloat16)
    vg_ref[...] = jnp.dot(xg_bf, wv_ref[...],
                          preferred_element_type=jnp.float32
                          ).astype(jnp.bfloat16)


def _tc_project(x2_bf, xg, wq_bf, wk_bf, wv_bf):
    s_len, d = x2_bf.shape
    nk = xg.shape[0]
    n_blk = 4
    xb, gb = s_len // n_blk, nk // n_blk
    return pl.pallas_call(
        _proj_body,
        grid=(n_blk,),
        in_specs=[
            pl.BlockSpec((xb, d), lambda i: (i, 0)),
            pl.BlockSpec((gb, d), lambda i: (i, 0)),
            pl.BlockSpec((d, d), lambda i: (0, 0)),
            pl.BlockSpec((d, d), lambda i: (0, 0)),
            pl.BlockSpec((d, d), lambda i: (0, 0)),
        ],
        out_specs=[
            pl.BlockSpec((xb, d), lambda i: (i, 0)),
            pl.BlockSpec((gb, d), lambda i: (i, 0)),
            pl.BlockSpec((gb, d), lambda i: (i, 0)),
        ],
        out_shape=[
            jax.ShapeDtypeStruct((s_len, d), jnp.bfloat16),
            jax.ShapeDtypeStruct((nk, d), jnp.bfloat16),
            jax.ShapeDtypeStruct((nk, d), jnp.bfloat16),
        ],
        compiler_params=pltpu.CompilerParams(
            dimension_semantics=("parallel",)),
    )(x2_bf, xg, wq_bf, wk_bf, wv_bf)


def _attention_body(q_ref, kg_ref, vg_ref, wo_ref, o_ref, *, heads, dh):
    ctx_parts = []
    for h in range(heads):
        lo = h * dh
        qh = q_ref[:, lo:lo + dh]
        kh = kg_ref[:, lo:lo + dh]
        vh = vg_ref[:, lo:lo + dh]
        s = lax.dot_general(qh, kh, (((1,), (1,)), ((), ())),
                            preferred_element_type=jnp.float32)
        m = jnp.max(s, axis=-1, keepdims=True)
        e = jnp.exp(s - m)
        denom = jnp.sum(e, axis=-1, keepdims=True)
        ctx_h = jnp.dot(e.astype(jnp.bfloat16), vh,
                        preferred_element_type=jnp.float32)
        ctx_parts.append(ctx_h / denom)
    ctx = jnp.concatenate(ctx_parts, axis=1).astype(jnp.bfloat16)
    o_ref[...] = jnp.dot(ctx, wo_ref[...], preferred_element_type=jnp.float32)


def _tc_attention(q_bf, kg_bf, vg_bf, wo_bf, heads, dh):
    s_len, d = q_bf.shape
    nk = kg_bf.shape[0]
    qb_rows = 512
    n_qb = s_len // qb_rows
    body = functools.partial(_attention_body, heads=heads, dh=dh)
    return pl.pallas_call(
        body,
        grid=(n_qb,),
        in_specs=[
            pl.BlockSpec((qb_rows, d), lambda i: (i, 0)),
            pl.BlockSpec((nk, d), lambda i: (0, 0)),
            pl.BlockSpec((nk, d), lambda i: (0, 0)),
            pl.BlockSpec((d, d), lambda i: (0, 0)),
        ],
        out_specs=pl.BlockSpec((qb_rows, d), lambda i: (i, 0)),
        out_shape=jax.ShapeDtypeStruct((s_len, d), jnp.float32),
        compiler_params=pltpu.CompilerParams(
            dimension_semantics=("parallel",)),
    )(q_bf, kg_bf, vg_bf, wo_bf)


def kernel(x, Wq, Wk, Wv, Wo, hilbert_map):
    b, s_len, d = x.shape
    heads = _NUM_HEADS
    dh = d // heads
    scale = 1.0 / math.sqrt(dh)
    x2 = x.reshape(s_len, d)
    x2_bf = x2.astype(jnp.bfloat16)
    wq_bf = (Wq * scale).astype(jnp.bfloat16)
    wk_bf = Wk.astype(jnp.bfloat16)
    wv_bf = Wv.astype(jnp.bfloat16)
    wo_bf = Wo.astype(jnp.bfloat16)
    kv_pos = lax.slice(hilbert_map, (0,), (s_len,), (_DILATION,))
    xg = _sc_gather_rows(x2, kv_pos)
    q_bf, kg_bf, vg_bf = _tc_project(x2_bf, xg, wq_bf, wk_bf, wv_bf)
    y = _tc_attention(q_bf, kg_bf, vg_bf, wo_bf, heads, dh)
    out2 = _sc_scatter_rows(y, hilbert_map)
    return out2.reshape(b, s_len, d)
